# Initial kernel scaffold; baseline (speedup 1.0000x reference)
import functools

import jax
import jax.numpy as jnp
from jax import lax
from jax.experimental import pallas as pl
from jax.experimental.pallas import tpu as pltpu
from jax.experimental.pallas import tpu_sc as plsc

_NC = 2
_NS = 16
_K = 80


def _seg_sum_sc(h, src, dst):
    N, D = h.shape
    E = src.shape[0]
    NW = _NC * _NS
    Et = E // NW
    NB = Et // _K
    RPT = N // _NS
    ZR = 125

    mesh = plsc.VectorSubcoreMesh(core_axis_name="c", subcore_axis_name="s")

    @functools.partial(
        pl.kernel,
        mesh=mesh,
        out_type=jax.ShapeDtypeStruct((_NC, N, D), jnp.float32),
        scratch_types=[
            pltpu.VMEM_SHARED((N, D), jnp.float32),
            pltpu.VMEM((ZR, D), jnp.float32),
            pltpu.VMEM((_K,), jnp.int32),
            pltpu.VMEM((_K,), jnp.int32),
            pltpu.VMEM((_K, D), jnp.float32),
            pltpu.SemaphoreType.DMA,
        ],
    )
    def seg_kernel(h_hbm, src_hbm, dst_hbm, out_hbm, acc, zbuf, idx_s, idx_d,
                   rows, sem):
        c = lax.axis_index("c")
        s = lax.axis_index("s")
        tid = c * _NS + s

        zv = jnp.zeros((16,), jnp.float32)

        def zrow(i, carry):
            def zcol(j, carry2):
                zbuf[i, pl.ds(j * 16, 16)] = zv
                return carry2
            return lax.fori_loop(0, D // 16, zcol, carry)
        lax.fori_loop(0, ZR, zrow, 0)

        def zcopy(k, carry):
            pltpu.sync_copy(zbuf, acc.at[pl.ds(s * RPT + k * ZR, ZR)])
            return carry
        lax.fori_loop(0, RPT // ZR, zcopy, 0)
        plsc.subcore_barrier()

        base = tid * Et

        def body(b, carry):
            off = base + b * _K
            pltpu.sync_copy(src_hbm.at[pl.ds(off, _K)], idx_s)
            pltpu.sync_copy(dst_hbm.at[pl.ds(off, _K)], idx_d)
            pltpu.async_copy(h_hbm.at[idx_s], rows, sem).wait()
            pltpu.sync_copy(rows, acc.at[idx_d], add=True)
            return carry
        lax.fori_loop(0, NB, body, 0)
        plsc.subcore_barrier()

        pltpu.sync_copy(acc.at[pl.ds(s * RPT, RPT)],
                        out_hbm.at[c, pl.ds(s * RPT, RPT)])

    return seg_kernel(h, src, dst)


def _mlp_tc(h, parts, eps, Wa, ba, Wb, bb, g, be, Wl=None, bl=None):
    N, D = h.shape
    final = Wl is not None
    out_dim = Wl.shape[1] if final else Wb.shape[1]

    def body(h_ref, p_ref, eps_ref, Wa_ref, ba_ref, Wb_ref, bb_ref, g_ref,
             be_ref, *rest):
        out_ref = rest[-1]
        z = h_ref[...] * (1.0 + eps_ref[0, 0]) + p_ref[0] + p_ref[1]
        z = jnp.dot(z, Wa_ref[...], preferred_element_type=jnp.float32)
        z = jnp.maximum(z + ba_ref[...], 0.0)
        z = jnp.dot(z, Wb_ref[...], preferred_element_type=jnp.float32)
        z = jnp.maximum(z + bb_ref[...], 0.0)
        mean = jnp.mean(z, axis=0, keepdims=True)
        zc = z - mean
        var = jnp.mean(zc * zc, axis=0, keepdims=True)
        z = zc * lax.rsqrt(var + 1e-5) * g_ref[...] + be_ref[...]
        z = jnp.maximum(z, 0.0)
        if final:
            Wl_ref, bl_ref = rest[0], rest[1]
            z = jnp.dot(z, Wl_ref[...], preferred_element_type=jnp.float32)
            z = z + bl_ref[...]
        out_ref[...] = z

    args = [h, parts, jnp.reshape(eps, (1, 1)), Wa, jnp.reshape(ba, (1, -1)),
            Wb, jnp.reshape(bb, (1, -1)), jnp.reshape(g, (1, -1)),
            jnp.reshape(be, (1, -1))]
    if final:
        args += [Wl, jnp.reshape(bl, (1, -1))]
    return pl.pallas_call(
        body,
        out_shape=jax.ShapeDtypeStruct((N, out_dim), jnp.float32),
    )(*args)


def kernel(x, edge_index, W1a, b1a, W1b, b1b, g1, be1, eps1,
           W2a, b2a, W2b, b2b, g2, be2, eps2,
           W3a, b3a, W3b, b3b, g3, be3, eps3, Wl, bl):
    src = edge_index[0]
    dst = edge_index[1]

    p1 = _seg_sum_sc(x, src, dst)
    h = _mlp_tc(x, p1, eps1, W1a, b1a, W1b, b1b, g1, be1)
    p2 = _seg_sum_sc(h, src, dst)
    h = _mlp_tc(h, p2, eps2, W2a, b2a, W2b, b2b, g2, be2)
    p3 = _seg_sum_sc(h, src, dst)
    h = _mlp_tc(h, p3, eps3, W3a, b3a, W3b, b3b, g3, be3, Wl=Wl, bl=bl)
    return h


# speedup vs baseline: 4.5746x; 4.5746x over previous
import functools

import jax
import jax.numpy as jnp
from jax import lax
from jax.experimental import pallas as pl
from jax.experimental.pallas import tpu as pltpu
from jax.experimental.pallas import tpu_sc as plsc

_NC = 2
_NS = 16
_K = 80


def _seg_sum_sc(h, src, dst):
    N, D = h.shape
    E = src.shape[0]
    NW = _NC * _NS
    Et = E // NW
    NB = Et // _K
    RPT = (N // _NS) // 8 * 8
    TAIL = N - RPT * _NS
    ZR = RPT // 2

    mesh = plsc.VectorSubcoreMesh(core_axis_name="c", subcore_axis_name="s")

    @functools.partial(
        pl.kernel,
        mesh=mesh,
        out_type=jax.ShapeDtypeStruct((_NC, N, D), jnp.float32),
        scratch_types=[
            pltpu.VMEM_SHARED((N, D), jnp.float32),
            pltpu.VMEM((ZR, D), jnp.float32),
            pltpu.VMEM((_K,), jnp.int32),
            pltpu.VMEM((_K,), jnp.int32),
            pltpu.VMEM((_K, D), jnp.float32),
            pltpu.SemaphoreType.DMA,
        ],
    )
    def seg_kernel(h_hbm, src_hbm, dst_hbm, out_hbm, acc, zbuf, idx_s, idx_d,
                   rows, sem):
        c = lax.axis_index("c")
        s = lax.axis_index("s")
        tid = c * _NS + s

        zv = jnp.zeros((16,), jnp.float32)

        def zrow(i, carry):
            def zcol(j, carry2):
                zbuf[i, pl.ds(j * 16, 16)] = zv
                return carry2
            return lax.fori_loop(0, D // 16, zcol, carry)
        lax.fori_loop(0, ZR, zrow, 0)

        def zcopy(k, carry):
            pltpu.sync_copy(zbuf, acc.at[pl.ds(s * RPT + k * ZR, ZR)])
            return carry
        lax.fori_loop(0, RPT // ZR, zcopy, 0)

        @pl.when(s == 0)
        def _():
            pltpu.sync_copy(zbuf.at[pl.ds(0, TAIL)],
                            acc.at[pl.ds(_NS * RPT, TAIL)])
        plsc.subcore_barrier()

        base = tid * Et

        def body(b, carry):
            off = base + b * _K
            pltpu.sync_copy(src_hbm.at[pl.ds(off, _K)], idx_s)
            pltpu.sync_copy(dst_hbm.at[pl.ds(off, _K)], idx_d)
            pltpu.async_copy(h_hbm.at[idx_s], rows, sem).wait()
            pltpu.sync_copy(rows, acc.at[idx_d], add=True)
            return carry
        lax.fori_loop(0, NB, body, 0)
        plsc.subcore_barrier()

        pltpu.sync_copy(acc.at[pl.ds(s * RPT, RPT)],
                        out_hbm.at[c, pl.ds(s * RPT, RPT)])

        @pl.when(s == 0)
        def _():
            pltpu.sync_copy(acc.at[pl.ds(_NS * RPT, TAIL)],
                            out_hbm.at[c, pl.ds(_NS * RPT, TAIL)])

    return seg_kernel(h, src, dst)


def _mlp_tc(h, parts, eps, Wa, ba, Wb, bb, g, be, Wl=None, bl=None):
    N, D = h.shape
    final = Wl is not None
    out_dim = Wl.shape[1] if final else Wb.shape[1]

    def body(h_ref, p_ref, eps_ref, Wa_ref, ba_ref, Wb_ref, bb_ref, g_ref,
             be_ref, *rest):
        out_ref = rest[-1]
        bf = jnp.bfloat16
        z = h_ref[...] * (1.0 + eps_ref[0, 0]) + p_ref[0] + p_ref[1]
        z = jnp.dot(z.astype(bf), Wa_ref[...].astype(bf),
                    preferred_element_type=jnp.float32)
        z = jnp.maximum(z + ba_ref[...], 0.0)
        z = jnp.dot(z.astype(bf), Wb_ref[...].astype(bf),
                    preferred_element_type=jnp.float32)
        z = jnp.maximum(z + bb_ref[...], 0.0)
        mean = jnp.mean(z, axis=0, keepdims=True)
        zc = z - mean
        var = jnp.mean(zc * zc, axis=0, keepdims=True)
        z = zc * lax.rsqrt(var + 1e-5) * g_ref[...] + be_ref[...]
        z = jnp.maximum(z, 0.0)
        if final:
            Wl_ref, bl_ref = rest[0], rest[1]
            z = jnp.dot(z.astype(bf), Wl_ref[...].astype(bf),
                        preferred_element_type=jnp.float32)
            z = z + bl_ref[...]
        out_ref[...] = z

    args = [h, parts, jnp.reshape(eps, (1, 1)), Wa, jnp.reshape(ba, (1, -1)),
            Wb, jnp.reshape(bb, (1, -1)), jnp.reshape(g, (1, -1)),
            jnp.reshape(be, (1, -1))]
    if final:
        args += [Wl, jnp.reshape(bl, (1, -1))]
    return pl.pallas_call(
        body,
        out_shape=jax.ShapeDtypeStruct((N, out_dim), jnp.float32),
    )(*args)


def kernel(x, edge_index, W1a, b1a, W1b, b1b, g1, be1, eps1,
           W2a, b2a, W2b, b2b, g2, be2, eps2,
           W3a, b3a, W3b, b3b, g3, be3, eps3, Wl, bl):
    src = edge_index[0]
    dst = edge_index[1]

    p1 = _seg_sum_sc(x, src, dst)
    h = _mlp_tc(x, p1, eps1, W1a, b1a, W1b, b1b, g1, be1)
    p2 = _seg_sum_sc(h, src, dst)
    h = _mlp_tc(h, p2, eps2, W2a, b2a, W2b, b2b, g2, be2)
    p3 = _seg_sum_sc(h, src, dst)
    h = _mlp_tc(h, p3, eps3, W3a, b3a, W3b, b3b, g3, be3, Wl=Wl, bl=bl)
    return h
